# baseline (speedup 1.0000x reference)
import functools

import jax
import jax.numpy as jnp
from jax import lax
from jax.experimental import pallas as pl
from jax.experimental.pallas import tpu as pltpu
from jax.experimental.pallas import tpu_sc as plsc

V = 1000
C = 20
VPAD = 1024
CPAD = 32
B = 4096
LSEQ = 200
NTILES = 32
ROWS_PER_TILE = B // NTILES
GROUPS = ROWS_PER_TILE // 16


def _bf16_bits(x):
    r = jax.lax.bitcast_convert_type(x, jnp.int32)
    rb = jax.lax.shift_right_logical(r, 16) & 1
    return jax.lax.shift_right_logical(r + 0x7FFF + rb, 16)


def _prep_body(tcc_t_ref, cc_ref, packed_ref, bias_ref):
    tcc = tcc_t_ref[...]
    colsum = jnp.sum(tcc, axis=1, keepdims=True)
    m = jnp.log(tcc) - jnp.log(colsum)
    col = lax.broadcasted_iota(jnp.int32, m.shape, 1)
    m = jnp.where(col == 0, 0.0, m)
    m8 = m.astype(jnp.float8_e4m3fn)
    m83 = jnp.reshape(m8, (C // 4, 4, V))

    def _byte(k):
        b = jax.lax.bitcast_convert_type(m83[:, k, :], jnp.int8)
        return jax.lax.convert_element_type(b, jnp.int32) & 0xFF

    word = _byte(0) | (_byte(1) << 8) | (_byte(2) << 16) | (_byte(3) << 24)
    packed_ref[:, :V] = word
    cc = cc_ref[...]
    bias_ref[...] = jnp.where(cc > 0, jnp.log(cc), -jnp.inf)


_prep = pl.pallas_call(
    _prep_body,
    out_shape=(
        jax.ShapeDtypeStruct((C // 4, VPAD), jnp.int32),
        jax.ShapeDtypeStruct((1, C), jnp.float32),
    ),
)


def _sc_mesh():
    return plsc.VectorSubcoreMesh(core_axis_name="c", subcore_axis_name="s")


@functools.partial(
    pl.kernel,
    mesh=_sc_mesh(),
    compiler_params=pltpu.CompilerParams(needs_layout_passes=False),
    out_type=jax.ShapeDtypeStruct((B, C), jnp.float32),
    scratch_types=[
        pltpu.VMEM(((C // 4) * VPAD,), jnp.int32),
        pltpu.VMEM((ROWS_PER_TILE, LSEQ), jnp.int32),
        pltpu.VMEM((1, C), jnp.float32),
        pltpu.VMEM((ROWS_PER_TILE, C), jnp.float32),
        pltpu.SemaphoreType.DMA,
        pltpu.SemaphoreType.DMA,
        pltpu.SemaphoreType.DMA,
    ],
)
def _nb_sc(text_t, table, bias, out, table_v, text_v, bias_v, out_v,
           sem_a, sem_b, sem_c):
    wid = lax.axis_index("s") * 2 + lax.axis_index("c")
    base = wid * ROWS_PER_TILE
    cp_a = pltpu.async_copy(table, table_v, sem_a)
    cp_c = pltpu.async_copy(bias, bias_v, sem_c)
    cp_b = pltpu.async_copy(
        text_t.at[pl.ds(base, ROWS_PER_TILE), :], text_v, sem_b
    )
    cp_a.wait()
    cp_c.wait()
    cp_b.wait()

    zero16 = jnp.zeros((16,), jnp.int32)
    biasv = tuple(
        plsc.load_gather(bias_v, [zero16, jnp.full((16,), c, jnp.int32)])
        for c in range(C)
    )

    lane = lax.iota(jnp.int32, 16)
    NE = 8
    NW = C // 4
    for g in range(GROUPS):
        rows16 = g * 16 + lane

        def e_body(e, accs):
            sub = [jnp.zeros((32,), jnp.bfloat16) for _ in range(2 * NW)]
            for u in range(NE):
                l = e * NE + u
                toks = plsc.load_gather(text_v, [rows16, zero16 + l])
                for w in range(NW):
                    word = plsc.load_gather(table_v, [toks + w * VPAD])
                    f8 = plsc.bitcast(word, jnp.float8_e4m3fn)
                    a, b = plsc.unpack(
                        f8,
                        format=plsc.PackFormat.INTERLEAVED,
                        preferred_element_type=jnp.bfloat16,
                    )
                    sub[2 * w] = sub[2 * w] + a
                    sub[2 * w + 1] = sub[2 * w + 1] + b
            new = list(accs)
            for w in range(NW):
                e0, e2 = plsc.unpack(sub[2 * w], format=plsc.PackFormat.INTERLEAVED)
                o1, o3 = plsc.unpack(sub[2 * w + 1], format=plsc.PackFormat.INTERLEAVED)
                new[4 * w] = new[4 * w] + e0
                new[4 * w + 2] = new[4 * w + 2] + e2
                new[4 * w + 1] = new[4 * w + 1] + o1
                new[4 * w + 3] = new[4 * w + 3] + o3
            return tuple(new)

        accs = lax.fori_loop(0, LSEQ // NE, e_body, biasv)
        rows = g * 16 + lane
        for c in range(C):
            plsc.store_scatter(
                out_v, [rows, jnp.full((16,), c, jnp.int32)], accs[c]
            )

    pltpu.sync_copy(out_v, out.at[pl.ds(base, ROWS_PER_TILE), :])


def kernel(text, token_class_counts, class_counts):
    tcc_t = jnp.transpose(token_class_counts)
    cc2 = jnp.reshape(class_counts, (1, C))
    packed, bias = _prep(tcc_t, cc2)
    table_flat = jnp.reshape(packed, ((C // 4) * VPAD,))
    return _nb_sc(text, table_flat, bias)


# speedup vs baseline: 1.3767x; 1.3767x over previous
import functools

import jax
import jax.numpy as jnp
from jax import lax
from jax.experimental import pallas as pl
from jax.experimental.pallas import tpu as pltpu
from jax.experimental.pallas import tpu_sc as plsc

V = 1000
C = 20
VPAD = 1024
CPAD = 32
B = 4096
LSEQ = 200
NTILES = 32
ROWS_PER_TILE = B // NTILES
GROUPS = ROWS_PER_TILE // 16


def _prep_body(tcc_t_ref, cc_ref, packed_ref, bias_ref):
    tcc = tcc_t_ref[...]
    colsum = jnp.sum(tcc, axis=1, keepdims=True)
    m = jnp.log(tcc) - jnp.log(colsum)
    col = lax.broadcasted_iota(jnp.int32, m.shape, 1)
    m = jnp.where(col == 0, 0.0, m)
    m8 = m.astype(jnp.float8_e4m3fn)
    m83 = jnp.reshape(m8, (C // 4, 4, V))

    def _byte(k):
        b = jax.lax.bitcast_convert_type(m83[:, k, :], jnp.int8)
        return jax.lax.convert_element_type(b, jnp.int32) & 0xFF

    word = _byte(0) | (_byte(1) << 8) | (_byte(2) << 16) | (_byte(3) << 24)
    packed_ref[:, :V] = word
    cc = cc_ref[...]
    bias_ref[...] = jnp.where(cc > 0, jnp.log(cc), -jnp.inf)


_prep = pl.pallas_call(
    _prep_body,
    out_shape=(
        jax.ShapeDtypeStruct((C // 4, VPAD), jnp.int32),
        jax.ShapeDtypeStruct((1, C), jnp.float32),
    ),
)


def _sc_mesh():
    return plsc.VectorSubcoreMesh(core_axis_name="c", subcore_axis_name="s")


@functools.partial(
    pl.kernel,
    mesh=_sc_mesh(),
    compiler_params=pltpu.CompilerParams(needs_layout_passes=False),
    out_type=jax.ShapeDtypeStruct((B, C), jnp.float32),
    scratch_types=[
        pltpu.VMEM(((C // 4) * VPAD,), jnp.int32),
        pltpu.VMEM((LSEQ, ROWS_PER_TILE), jnp.int32),
        pltpu.VMEM((1, C), jnp.float32),
        pltpu.VMEM((ROWS_PER_TILE, C), jnp.float32),
        pltpu.SemaphoreType.DMA,
        pltpu.SemaphoreType.DMA,
        pltpu.SemaphoreType.DMA,
    ],
)
def _nb_sc(text_t, table, bias, out, table_v, text_v, bias_v, out_v,
           sem_a, sem_b, sem_c):
    wid = lax.axis_index("s") * 2 + lax.axis_index("c")
    base = wid * ROWS_PER_TILE
    cp_a = pltpu.async_copy(table, table_v, sem_a)
    cp_c = pltpu.async_copy(bias, bias_v, sem_c)
    cp_b = pltpu.async_copy(
        text_t.at[:, pl.ds(base, ROWS_PER_TILE)], text_v, sem_b
    )
    cp_a.wait()
    cp_c.wait()
    cp_b.wait()

    zero16 = jnp.zeros((16,), jnp.int32)
    biasv = tuple(
        plsc.load_gather(bias_v, [zero16, jnp.full((16,), c, jnp.int32)])
        for c in range(C)
    )

    lane = lax.iota(jnp.int32, 16)
    NE = 8
    NW = C // 4
    for g in range(GROUPS):
        def e_body(e, accs):
            sub = [jnp.zeros((32,), jnp.bfloat16) for _ in range(2 * NW)]
            for u in range(NE):
                l = e * NE + u
                toks = text_v[l, pl.ds(g * 16, 16)]
                for w in range(NW):
                    word = plsc.load_gather(table_v, [toks + w * VPAD])
                    f8 = plsc.bitcast(word, jnp.float8_e4m3fn)
                    a, b = plsc.unpack(
                        f8,
                        format=plsc.PackFormat.INTERLEAVED,
                        preferred_element_type=jnp.bfloat16,
                    )
                    sub[2 * w] = sub[2 * w] + a
                    sub[2 * w + 1] = sub[2 * w + 1] + b
            new = list(accs)
            for w in range(NW):
                e0, e2 = plsc.unpack(sub[2 * w], format=plsc.PackFormat.INTERLEAVED)
                o1, o3 = plsc.unpack(sub[2 * w + 1], format=plsc.PackFormat.INTERLEAVED)
                new[4 * w] = new[4 * w] + e0
                new[4 * w + 2] = new[4 * w + 2] + e2
                new[4 * w + 1] = new[4 * w + 1] + o1
                new[4 * w + 3] = new[4 * w + 3] + o3
            return tuple(new)

        accs = lax.fori_loop(0, LSEQ // NE, e_body, biasv)
        rows = g * 16 + lane
        for c in range(C):
            plsc.store_scatter(
                out_v, [rows, jnp.full((16,), c, jnp.int32)], accs[c]
            )

    pltpu.sync_copy(out_v, out.at[pl.ds(base, ROWS_PER_TILE), :])


def kernel(text, token_class_counts, class_counts):
    tcc_t = jnp.transpose(token_class_counts)
    cc2 = jnp.reshape(class_counts, (1, C))
    packed, bias = _prep(tcc_t, cc2)
    table_flat = jnp.reshape(packed, ((C // 4) * VPAD,))
    text_t = jnp.transpose(text)
    return _nb_sc(text_t, table_flat, bias)
